# baseline (speedup 1.0000x reference)
import functools

import jax
import jax.numpy as jnp
from jax import lax
from jax.experimental import pallas as pl
from jax.experimental.pallas import tpu as pltpu
from jax.experimental.pallas import tpu_sc as plsc

_G = 128
_K = 32
_EPS = 1e-5
_INF = 1e30


def _fps_knn_body(px_ref, py_ref, pz_ref, cen_ref, idx_ref,
                  cxs_ref, cys_ref, czs_ref):
    px = px_ref[...]
    py = py_ref[...]
    pz = pz_ref[...]
    B, N = px.shape
    iota_n = lax.broadcasted_iota(jnp.int32, (1, N), 1)

    def fps_step(i, carry):
        dists, far = carry
        m = iota_n == far
        zx = jnp.zeros((B, N), jnp.float32)
        cx = jnp.sum(jnp.where(m, px, zx), axis=1)
        cy = jnp.sum(jnp.where(m, py, zx), axis=1)
        cz = jnp.sum(jnp.where(m, pz, zx), axis=1)
        cxs_ref[pl.ds(i, 1), :] = cx[None, :]
        cys_ref[pl.ds(i, 1), :] = cy[None, :]
        czs_ref[pl.ds(i, 1), :] = cz[None, :]
        dx = px - cx[:, None]
        dy = py - cy[:, None]
        dz = pz - cz[:, None]
        d = dx * dx + dy * dy + dz * dz
        dists = jnp.minimum(dists, d)
        far = jnp.argmax(dists, axis=1).astype(jnp.int32)[:, None]
        return dists, far

    lax.fori_loop(0, _G, fps_step,
                  (jnp.full((B, N), 1e10, dtype=jnp.float32),
                   jnp.zeros((B, 1), jnp.int32)))

    cx = jnp.swapaxes(cxs_ref[...], 0, 1)
    cy = jnp.swapaxes(cys_ref[...], 0, 1)
    cz = jnp.swapaxes(czs_ref[...], 0, 1)
    cen_ref[...] = jnp.concatenate(
        [cx[:, :, None], cy[:, :, None], cz[:, :, None],
         jnp.zeros((B, _G, 13), jnp.float32)], axis=-1)

    CHUNK = 16
    iota3 = lax.broadcasted_iota(jnp.int32, (1, 1, N), 2)
    for b0 in range(0, B, CHUNK):
        bs = slice(b0, b0 + CHUNK)
        ddx = cx[bs][:, :, None] - px[bs][:, None, :]
        ddy = cy[bs][:, :, None] - py[bs][:, None, :]
        ddz = cz[bs][:, :, None] - pz[bs][:, None, :]
        d0 = ddx * ddx + ddy * ddy + ddz * ddz
        key0 = (lax.bitcast_convert_type(d0, jnp.int32) & ~jnp.int32(2047)
                ) | iota3

        def knn_step(k, key):
            kmin = jnp.min(key, axis=2)
            a = kmin & jnp.int32(2047)
            idx_ref[bs, pl.ds(k, 1), :] = a[:, None, :]
            return jnp.where(key == kmin[:, :, None],
                             jnp.int32(0x7FFFFFFF), key)

        lax.fori_loop(0, _K, knn_step, key0)


def _fps_knn(px, py, pz):
    B, N = px.shape
    return pl.pallas_call(
        _fps_knn_body,
        out_shape=(jax.ShapeDtypeStruct((B, _G, 16), jnp.float32),
                   jax.ShapeDtypeStruct((B, _K, _G), jnp.int32)),
        scratch_shapes=[pltpu.VMEM((_G, B), jnp.float32)] * 3,
    )(px, py, pz)


def _sc_gather_nb(px, py, pz, idx):
    B, N = px.shape
    M = idx.shape[0]
    info = plsc.get_sparse_core_info()
    NC, NS = info.num_cores, info.num_subcores
    NW = NC * NS
    per_w = M // NW
    wpb = NW // B
    mesh = plsc.VectorSubcoreMesh(core_axis_name="c", subcore_axis_name="s")

    @functools.partial(
        pl.kernel, mesh=mesh,
        compiler_params=pltpu.CompilerParams(needs_layout_passes=False),
        out_type=(jax.ShapeDtypeStruct((M,), jnp.float32),
                  jax.ShapeDtypeStruct((M,), jnp.float32),
                  jax.ShapeDtypeStruct((M,), jnp.float32)),
        scratch_types=[
            pltpu.VMEM((N,), jnp.float32),
            pltpu.VMEM((N,), jnp.float32),
            pltpu.VMEM((N,), jnp.float32),
            pltpu.VMEM((per_w,), jnp.int32),
            pltpu.VMEM((per_w,), jnp.float32),
            pltpu.VMEM((per_w,), jnp.float32),
            pltpu.VMEM((per_w,), jnp.float32),
        ],
    )
    def k(px_hbm, py_hbm, pz_hbm, idx_hbm, ox_hbm, oy_hbm, oz_hbm,
          tx, ty, tz, idx_v, gx, gy, gz):
        wid = lax.axis_index("s") * NC + lax.axis_index("c")
        base = wid * per_w
        b = wid // wpb
        pltpu.sync_copy(px_hbm.at[b], tx)
        pltpu.sync_copy(py_hbm.at[b], ty)
        pltpu.sync_copy(pz_hbm.at[b], tz)
        pltpu.sync_copy(idx_hbm.at[pl.ds(base, per_w)], idx_v)

        def body(j, carry):
            o = j * 16
            iv = idx_v[pl.ds(o, 16)]
            gx[pl.ds(o, 16)] = plsc.load_gather(tx, [iv])
            gy[pl.ds(o, 16)] = plsc.load_gather(ty, [iv])
            gz[pl.ds(o, 16)] = plsc.load_gather(tz, [iv])
            return carry

        lax.fori_loop(0, per_w // 16, body, 0)
        pltpu.sync_copy(gx, ox_hbm.at[pl.ds(base, per_w)])
        pltpu.sync_copy(gy, oy_hbm.at[pl.ds(base, per_w)])
        pltpu.sync_copy(gz, oz_hbm.at[pl.ds(base, per_w)])

    return k(px, py, pz, idx)


def _sc_gather_rows(table, idx, D):
    info = plsc.get_sparse_core_info()
    NC, NS = info.num_cores, info.num_subcores
    NW = NC * NS
    M = idx.shape[0]
    b_per_w = M // NW
    mesh = plsc.VectorSubcoreMesh(core_axis_name="c", subcore_axis_name="s")

    @functools.partial(
        pl.kernel, mesh=mesh,
        out_type=jax.ShapeDtypeStruct((M, D), jnp.float32),
        scratch_types=[
            pltpu.VMEM((b_per_w,), jnp.int32),
            pltpu.VMEM((b_per_w, D), jnp.float32),
            pltpu.SemaphoreType.DMA,
        ],
    )
    def k(table_hbm, idx_hbm, out_hbm, idx_v, rows_v, sem):
        wid = lax.axis_index("s") * NC + lax.axis_index("c")
        base = wid * b_per_w
        pltpu.sync_copy(idx_hbm.at[pl.ds(base, b_per_w)], idx_v)
        pltpu.async_copy(table_hbm.at[idx_v], rows_v, sem).wait()
        pltpu.sync_copy(rows_v, out_hbm.at[pl.ds(base, b_per_w)])

    return k(table, idx)


def _stats_body(nbx_ref, nby_ref, nbz_ref, cen_ref, w1_ref, b1_ref, g1_ref,
                be1_ref, s_ref, t_ref, *, M):
    gx = nbx_ref[...] - cen_ref[:, 0:1]
    gy = nby_ref[...] - cen_ref[:, 1:2]
    gz = nbz_ref[...] - cen_ref[:, 2:3]
    mx = jnp.sum(gx) / M
    my = jnp.sum(gy) / M
    mz = jnp.sum(gz) / M
    axx = jnp.sum(gx * gx) / M
    ayy = jnp.sum(gy * gy) / M
    azz = jnp.sum(gz * gz) / M
    axy = jnp.sum(gx * gy) / M
    axz = jnp.sum(gx * gz) / M
    ayz = jnp.sum(gy * gz) / M
    w0 = w1_ref[0:1, :]
    w1 = w1_ref[1:2, :]
    w2 = w1_ref[2:3, :]
    b1 = b1_ref[...]
    mu0 = mx * w0 + my * w1 + mz * w2
    mu = mu0 + b1
    ex2 = (axx * w0 * w0 + ayy * w1 * w1 + azz * w2 * w2
           + 2.0 * (axy * w0 * w1 + axz * w0 * w2 + ayz * w1 * w2)
           + 2.0 * b1 * mu0 + b1 * b1)
    var = ex2 - mu * mu
    s = g1_ref[...] * lax.rsqrt(var + _EPS)
    s_ref[...] = s
    t_ref[...] = be1_ref[...] - mu * s


def _bn1_stats(nbx, nby, nbz, cen, W1, b1, g1, be1):
    M = nbx.shape[0] * nbx.shape[1]
    return pl.pallas_call(
        functools.partial(_stats_body, M=float(M)),
        out_shape=(jax.ShapeDtypeStruct((1, 128), jnp.float32),
                   jax.ShapeDtypeStruct((1, 128), jnp.float32)),
    )(nbx, nby, nbz, cen, W1, b1, g1, be1)


def _enc1_body(nbx_ref, nby_ref, nbz_ref, cen_ref, w1_ref, s1_ref, t1_ref,
               w2_ref, b2_ref, w3_ref, b3_ref, h3_ref, sums_ref):
    Gb, Kn = nbx_ref.shape
    R = Gb * Kn
    gx = nbx_ref[...] - cen_ref[:, 0:1]
    gy = nby_ref[...] - cen_ref[:, 1:2]
    gz = nbz_ref[...] - cen_ref[:, 2:3]
    w0 = w1_ref[0:1, :].reshape(1, 1, 128)
    w1 = w1_ref[1:2, :].reshape(1, 1, 128)
    w2 = w1_ref[2:3, :].reshape(1, 1, 128)
    h1 = (gx[:, :, None] * w0 + gy[:, :, None] * w1
          + gz[:, :, None] * w2).reshape(R, 128)
    a1 = jnp.maximum(h1 * s1_ref[...] + t1_ref[...], 0.0)
    h2 = jnp.dot(a1.astype(jnp.bfloat16), w2_ref[...],
                 preferred_element_type=jnp.float32) + b2_ref[...]
    fg = jnp.max(h2.reshape(Gb, Kn, 256), axis=1)
    fgb = jnp.broadcast_to(fg[:, None, :], (Gb, Kn, 256)).reshape(R, 256)
    cat = jnp.concatenate([fgb, h2], axis=1)
    h3 = jnp.dot(cat.astype(jnp.bfloat16), w3_ref[...],
                 preferred_element_type=jnp.float32) + b3_ref[...]
    h3_ref[...] = h3
    ssum = jnp.sum(h3, axis=0, keepdims=True)
    ssq = jnp.sum(h3 * h3, axis=0, keepdims=True)
    acc = jnp.concatenate([ssum, ssq], axis=0)
    @pl.when(pl.program_id(0) == 0)
    def _():
        sums_ref[...] = acc
    @pl.when(pl.program_id(0) != 0)
    def _():
        sums_ref[...] += acc


def _encoder1(nbx, nby, nbz, cen, W1, s1, t1, W2, b2, W3, b3):
    BG = nbx.shape[0]
    M = BG * _K
    GB = 32
    n = BG // GB
    full = lambda shp: pl.BlockSpec(shp, lambda i: (0, 0))
    blk = lambda shp: pl.BlockSpec(shp, lambda i: (i, 0))
    return pl.pallas_call(
        _enc1_body,
        grid=(n,),
        in_specs=[
            blk((GB, _K)), blk((GB, _K)), blk((GB, _K)), blk((GB, 16)),
            full((3, 128)), full((1, 128)), full((1, 128)),
            full((128, 256)), full((1, 256)),
            full((512, 512)), full((1, 512)),
        ],
        out_specs=(pl.BlockSpec((GB * _K, 512), lambda i: (i, 0)),
                   pl.BlockSpec((2, 512), lambda i: (0, 0))),
        out_shape=(jax.ShapeDtypeStruct((M, 512), jnp.float32),
                   jax.ShapeDtypeStruct((2, 512), jnp.float32)),
    )(nbx, nby, nbz, cen, W1, s1, t1, W2, b2, W3, b3)


def _enc2_body(h3_ref, sums_ref, g2_ref, be2_ref, w4_ref, b4_ref, feat_ref,
               *, M):
    h3 = h3_ref[...]
    R = h3.shape[0]
    Gb = R // _K
    mu = sums_ref[0:1, :] / M
    var = sums_ref[1:2, :] / M - mu * mu
    s2 = g2_ref[...] * lax.rsqrt(var + _EPS)
    t2 = be2_ref[...] - mu * s2
    a3 = jnp.maximum(h3 * s2 + t2, 0.0)
    h4 = jnp.dot(a3.astype(jnp.bfloat16), w4_ref[...],
                 preferred_element_type=jnp.float32) + b4_ref[...]
    feat_ref[...] = jnp.max(h4.reshape(Gb, _K, 256), axis=1)


def _encoder2(h3, sums, g2, be2, W4, b4):
    M = h3.shape[0]
    RB = 1024
    GB = RB // _K
    n = M // RB
    full = lambda shp: pl.BlockSpec(shp, lambda i: (0, 0))
    return pl.pallas_call(
        functools.partial(_enc2_body, M=float(M)),
        grid=(n,),
        in_specs=[
            pl.BlockSpec((RB, 512), lambda i: (i, 0)),
            full((2, 512)), full((1, 512)), full((1, 512)),
            full((512, 256)), full((1, 256)),
        ],
        out_specs=pl.BlockSpec((GB, 256), lambda i: (i, 0)),
        out_shape=jax.ShapeDtypeStruct((M // _K, 256), jnp.float32),
    )(h3, sums, g2, be2, W4, b4)


def _vq_body(feat_ref, cbt_ref, minv_ref, tok_ref):
    TB = cbt_ref.shape[1]
    cbt = cbt_ref[...]
    cn = jnp.sum(cbt * cbt, axis=0, keepdims=True)
    sc = jnp.dot(feat_ref[...], cbt,
                 preferred_element_type=jnp.float32)
    d2 = cn - 2.0 * sc
    m = jnp.min(d2, axis=1, keepdims=True)
    iota_t = lax.broadcasted_iota(jnp.int32, d2.shape, 1)
    a = jnp.min(jnp.where(d2 == m, iota_t, jnp.int32(2**30)), axis=1,
                keepdims=True) + pl.program_id(0) * TB
    @pl.when(pl.program_id(0) == 0)
    def _():
        minv_ref[...] = m
        tok_ref[...] = a
    @pl.when(pl.program_id(0) != 0)
    def _():
        better = m < minv_ref[...]
        minv_ref[...] = jnp.where(better, m, minv_ref[...])
        tok_ref[...] = jnp.where(better, a, tok_ref[...])


def _vq(feat, cbT):
    BG = feat.shape[0]
    T = cbT.shape[1]
    TB = 512
    return pl.pallas_call(
        _vq_body,
        grid=(T // TB,),
        in_specs=[
            pl.BlockSpec((BG, 256), lambda i: (0, 0)),
            pl.BlockSpec((256, TB), lambda i: (0, i)),
        ],
        out_specs=(pl.BlockSpec((BG, 1), lambda i: (0, 0)),
                   pl.BlockSpec((BG, 1), lambda i: (0, 0))),
        out_shape=(jax.ShapeDtypeStruct((BG, 1), jnp.float32),
                   jax.ShapeDtypeStruct((BG, 1), jnp.int32)),
    )(feat, cbT)


def _dec_body(q_ref, nbx_ref, nby_ref, nbz_ref, cen_ref, d1_ref, db1_ref,
              d2_ref, db2_ref, d3x_ref, d3y_ref, d3z_ref, db3x_ref,
              db3y_ref, db3z_ref, out_ref, *, denom):
    q = q_ref[...]
    h = jnp.maximum(jnp.dot(q, d1_ref[...], preferred_element_type=jnp.float32)
                    + db1_ref[...], 0.0)
    h = jnp.maximum(jnp.dot(h, d2_ref[...], preferred_element_type=jnp.float32)
                    + db2_ref[...], 0.0)
    rx = jnp.dot(h, d3x_ref[...], preferred_element_type=jnp.float32) + db3x_ref[...]
    ry = jnp.dot(h, d3y_ref[...], preferred_element_type=jnp.float32) + db3y_ref[...]
    rz = jnp.dot(h, d3z_ref[...], preferred_element_type=jnp.float32) + db3z_ref[...]
    gx = nbx_ref[...] - cen_ref[:, 0:1]
    gy = nby_ref[...] - cen_ref[:, 1:2]
    gz = nbz_ref[...] - cen_ref[:, 2:3]
    tx = rx[:, :, None] - gx[:, None, :]
    ty = ry[:, :, None] - gy[:, None, :]
    tz = rz[:, :, None] - gz[:, None, :]
    dd = tx * tx + ty * ty + tz * tz
    part = (jnp.sum(jnp.min(dd, axis=2)) + jnp.sum(jnp.min(dd, axis=1))) / denom
    part = jnp.reshape(part, (1, 1))
    @pl.when(pl.program_id(0) == 0)
    def _():
        out_ref[...] = part
    @pl.when(pl.program_id(0) != 0)
    def _():
        out_ref[...] += part


def _decode_chamfer(q, nbx, nby, nbz, cen, D1, db1, D2, db2,
                    D3x, D3y, D3z, db3x, db3y, db3z):
    BG = q.shape[0]
    GB = 256
    n = BG // GB
    denom = float(BG * _K)
    full = lambda shp: pl.BlockSpec(shp, lambda i: (0, 0))
    blk = lambda r: pl.BlockSpec(r, lambda i: (i, 0))
    return pl.pallas_call(
        functools.partial(_dec_body, denom=denom),
        grid=(n,),
        in_specs=[
            blk((GB, 256)), blk((GB, _K)), blk((GB, _K)), blk((GB, _K)),
            blk((GB, 16)),
            full((256, 512)), full((1, 512)),
            full((512, 256)), full((1, 256)),
            full((256, _K)), full((256, _K)), full((256, _K)),
            full((1, _K)), full((1, _K)), full((1, _K)),
        ],
        out_specs=pl.BlockSpec((1, 1), lambda i: (0, 0)),
        out_shape=jax.ShapeDtypeStruct((1, 1), jnp.float32),
    )(q, nbx, nby, nbz, cen, D1, db1, D2, db2,
      D3x, D3y, D3z, db3x, db3y, db3z)


def kernel(pts, W1, b1, g1, be1, W2, b2, W3, b3, g2, be2, W4, b4,
           codebook, D1, db1, D2, db2, D3, db3):
    B, N, _ = pts.shape
    C = W4.shape[1]
    BG = B * _G
    M = BG * _K

    px = pts[:, :, 0]
    py = pts[:, :, 1]
    pz = pts[:, :, 2]
    cen16, knn_kg = _fps_knn(px, py, pz)
    knn = knn_kg.transpose(0, 2, 1)

    nbx, nby, nbz = _sc_gather_nb(px, py, pz, knn.reshape(M))
    nbx = nbx.reshape(BG, _K)
    nby = nby.reshape(BG, _K)
    nbz = nbz.reshape(BG, _K)

    cen = cen16.reshape(BG, 16)
    r = lambda v: v.reshape(1, -1)
    s1, t1 = _bn1_stats(nbx, nby, nbz, cen, W1, r(b1), r(g1), r(be1))
    bf = jnp.bfloat16
    h3, sums = _encoder1(nbx, nby, nbz, cen, W1, s1, t1,
                         W2.astype(bf), r(b2), W3.astype(bf), r(b3))
    feat = _encoder2(h3, sums, r(g2), r(be2), W4.astype(bf), r(b4))
    _, tok = _vq(feat, codebook.T)
    q = _sc_gather_rows(codebook, tok.reshape(BG), C)

    loss = _decode_chamfer(
        q, nbx, nby, nbz, cen, D1, r(db1), D2, r(db2),
        D3[:, 0::3], D3[:, 1::3], D3[:, 2::3],
        r(db3[0::3]), r(db3[1::3]), r(db3[2::3]))
    return loss[0, 0]


# speedup vs baseline: 1.0211x; 1.0211x over previous
import functools

import jax
import jax.numpy as jnp
from jax import lax
from jax.experimental import pallas as pl
from jax.experimental.pallas import tpu as pltpu
from jax.experimental.pallas import tpu_sc as plsc

_G = 128
_K = 32
_EPS = 1e-5
_INF = 1e30


def _fps_knn_body(px_ref, py_ref, pz_ref, cen_ref, idx_ref,
                  cxs_ref, cys_ref, czs_ref):
    px = px_ref[...]
    py = py_ref[...]
    pz = pz_ref[...]
    B, N = px.shape
    iota_n = lax.broadcasted_iota(jnp.int32, (1, N), 1)

    def fps_step(i, carry):
        dists, far = carry
        m = iota_n == far
        zx = jnp.zeros((B, N), jnp.float32)
        cx = jnp.sum(jnp.where(m, px, zx), axis=1)
        cy = jnp.sum(jnp.where(m, py, zx), axis=1)
        cz = jnp.sum(jnp.where(m, pz, zx), axis=1)
        cxs_ref[pl.ds(i, 1), :] = cx[None, :]
        cys_ref[pl.ds(i, 1), :] = cy[None, :]
        czs_ref[pl.ds(i, 1), :] = cz[None, :]
        dx = px - cx[:, None]
        dy = py - cy[:, None]
        dz = pz - cz[:, None]
        d = dx * dx + dy * dy + dz * dz
        dists = jnp.minimum(dists, d)
        far = jnp.argmax(dists, axis=1).astype(jnp.int32)[:, None]
        return dists, far

    lax.fori_loop(0, _G, fps_step,
                  (jnp.full((B, N), 1e10, dtype=jnp.float32),
                   jnp.zeros((B, 1), jnp.int32)))

    cx = jnp.swapaxes(cxs_ref[...], 0, 1)
    cy = jnp.swapaxes(cys_ref[...], 0, 1)
    cz = jnp.swapaxes(czs_ref[...], 0, 1)
    cen_ref[...] = jnp.concatenate(
        [cx[:, :, None], cy[:, :, None], cz[:, :, None],
         jnp.zeros((B, _G, 13), jnp.float32)], axis=-1)

    CHUNK = 16
    iota3 = lax.broadcasted_iota(jnp.int32, (1, 1, N), 2)
    for b0 in range(0, B, CHUNK):
        bs = slice(b0, b0 + CHUNK)
        ddx = cx[bs][:, :, None] - px[bs][:, None, :]
        ddy = cy[bs][:, :, None] - py[bs][:, None, :]
        ddz = cz[bs][:, :, None] - pz[bs][:, None, :]
        d0 = ddx * ddx + ddy * ddy + ddz * ddz
        key0 = (lax.bitcast_convert_type(d0, jnp.int32) & ~jnp.int32(2047)
                ) | iota3

        def knn_step(k, key):
            kmin = jnp.min(key, axis=2)
            a = kmin & jnp.int32(2047)
            idx_ref[bs, pl.ds(k, 1), :] = a[:, None, :]
            return jnp.where(key == kmin[:, :, None],
                             jnp.int32(0x7FFFFFFF), key)

        lax.fori_loop(0, _K, knn_step, key0)


def _fps_knn(px, py, pz):
    B, N = px.shape
    return pl.pallas_call(
        _fps_knn_body,
        out_shape=(jax.ShapeDtypeStruct((B, _G, 16), jnp.float32),
                   jax.ShapeDtypeStruct((B, _K, _G), jnp.int32)),
        scratch_shapes=[pltpu.VMEM((_G, B), jnp.float32)] * 3,
    )(px, py, pz)


def _sc_gather_nb(px, py, pz, idx):
    B, N = px.shape
    M = idx.shape[0]
    info = plsc.get_sparse_core_info()
    NC, NS = info.num_cores, info.num_subcores
    NW = NC * NS
    per_w = M // NW
    wpb = NW // B
    mesh = plsc.VectorSubcoreMesh(core_axis_name="c", subcore_axis_name="s")

    @functools.partial(
        pl.kernel, mesh=mesh,
        compiler_params=pltpu.CompilerParams(needs_layout_passes=False),
        out_type=(jax.ShapeDtypeStruct((M,), jnp.float32),
                  jax.ShapeDtypeStruct((M,), jnp.float32),
                  jax.ShapeDtypeStruct((M,), jnp.float32)),
        scratch_types=[
            pltpu.VMEM((N,), jnp.float32),
            pltpu.VMEM((N,), jnp.float32),
            pltpu.VMEM((N,), jnp.float32),
            pltpu.VMEM((per_w,), jnp.int32),
            pltpu.VMEM((per_w,), jnp.float32),
            pltpu.VMEM((per_w,), jnp.float32),
            pltpu.VMEM((per_w,), jnp.float32),
        ],
    )
    def k(px_hbm, py_hbm, pz_hbm, idx_hbm, ox_hbm, oy_hbm, oz_hbm,
          tx, ty, tz, idx_v, gx, gy, gz):
        wid = lax.axis_index("s") * NC + lax.axis_index("c")
        base = wid * per_w
        b = wid // wpb
        pltpu.sync_copy(px_hbm.at[b], tx)
        pltpu.sync_copy(py_hbm.at[b], ty)
        pltpu.sync_copy(pz_hbm.at[b], tz)
        pltpu.sync_copy(idx_hbm.at[pl.ds(base, per_w)], idx_v)

        def body(j, carry):
            o = j * 16
            iv = idx_v[pl.ds(o, 16)]
            gx[pl.ds(o, 16)] = plsc.load_gather(tx, [iv])
            gy[pl.ds(o, 16)] = plsc.load_gather(ty, [iv])
            gz[pl.ds(o, 16)] = plsc.load_gather(tz, [iv])
            return carry

        lax.fori_loop(0, per_w // 16, body, 0)
        pltpu.sync_copy(gx, ox_hbm.at[pl.ds(base, per_w)])
        pltpu.sync_copy(gy, oy_hbm.at[pl.ds(base, per_w)])
        pltpu.sync_copy(gz, oz_hbm.at[pl.ds(base, per_w)])

    return k(px, py, pz, idx)


def _sc_gather_rows(table, idx, D):
    info = plsc.get_sparse_core_info()
    NC, NS = info.num_cores, info.num_subcores
    NW = NC * NS
    M = idx.shape[0]
    b_per_w = M // NW
    mesh = plsc.VectorSubcoreMesh(core_axis_name="c", subcore_axis_name="s")

    @functools.partial(
        pl.kernel, mesh=mesh,
        out_type=jax.ShapeDtypeStruct((M, D), jnp.float32),
        scratch_types=[
            pltpu.VMEM((b_per_w,), jnp.int32),
            pltpu.VMEM((b_per_w, D), jnp.float32),
            pltpu.SemaphoreType.DMA,
        ],
    )
    def k(table_hbm, idx_hbm, out_hbm, idx_v, rows_v, sem):
        wid = lax.axis_index("s") * NC + lax.axis_index("c")
        base = wid * b_per_w
        pltpu.sync_copy(idx_hbm.at[pl.ds(base, b_per_w)], idx_v)
        pltpu.async_copy(table_hbm.at[idx_v], rows_v, sem).wait()
        pltpu.sync_copy(rows_v, out_hbm.at[pl.ds(base, b_per_w)])

    return k(table, idx)


def _stats_body(nbx_ref, nby_ref, nbz_ref, cen_ref, w1_ref, b1_ref, g1_ref,
                be1_ref, s_ref, t_ref, *, M):
    gx = nbx_ref[...] - cen_ref[:, 0:1]
    gy = nby_ref[...] - cen_ref[:, 1:2]
    gz = nbz_ref[...] - cen_ref[:, 2:3]
    mx = jnp.sum(gx) / M
    my = jnp.sum(gy) / M
    mz = jnp.sum(gz) / M
    axx = jnp.sum(gx * gx) / M
    ayy = jnp.sum(gy * gy) / M
    azz = jnp.sum(gz * gz) / M
    axy = jnp.sum(gx * gy) / M
    axz = jnp.sum(gx * gz) / M
    ayz = jnp.sum(gy * gz) / M
    w0 = w1_ref[0:1, :]
    w1 = w1_ref[1:2, :]
    w2 = w1_ref[2:3, :]
    b1 = b1_ref[...]
    mu0 = mx * w0 + my * w1 + mz * w2
    mu = mu0 + b1
    ex2 = (axx * w0 * w0 + ayy * w1 * w1 + azz * w2 * w2
           + 2.0 * (axy * w0 * w1 + axz * w0 * w2 + ayz * w1 * w2)
           + 2.0 * b1 * mu0 + b1 * b1)
    var = ex2 - mu * mu
    s = g1_ref[...] * lax.rsqrt(var + _EPS)
    s_ref[...] = s
    t_ref[...] = be1_ref[...] - mu * s


def _bn1_stats(nbx, nby, nbz, cen, W1, b1, g1, be1):
    M = nbx.shape[0] * nbx.shape[1]
    return pl.pallas_call(
        functools.partial(_stats_body, M=float(M)),
        out_shape=(jax.ShapeDtypeStruct((1, 128), jnp.float32),
                   jax.ShapeDtypeStruct((1, 128), jnp.float32)),
    )(nbx, nby, nbz, cen, W1, b1, g1, be1)


def _enc1_body(nbx_ref, nby_ref, nbz_ref, cen_ref, w1_ref, s1_ref, t1_ref,
               w2_ref, b2_ref, w3_ref, b3_ref, h3_ref, sums_ref):
    Gb, Kn = nbx_ref.shape
    R = Gb * Kn
    gx = nbx_ref[...] - cen_ref[:, 0:1]
    gy = nby_ref[...] - cen_ref[:, 1:2]
    gz = nbz_ref[...] - cen_ref[:, 2:3]
    w0 = w1_ref[0:1, :].reshape(1, 1, 128)
    w1 = w1_ref[1:2, :].reshape(1, 1, 128)
    w2 = w1_ref[2:3, :].reshape(1, 1, 128)
    h1 = (gx[:, :, None] * w0 + gy[:, :, None] * w1
          + gz[:, :, None] * w2).reshape(R, 128)
    a1 = jnp.maximum(h1 * s1_ref[...] + t1_ref[...], 0.0)
    h2 = jnp.dot(a1, w2_ref[...], preferred_element_type=jnp.float32) + b2_ref[...]
    fg = jnp.max(h2.reshape(Gb, Kn, 256), axis=1)
    fgb = jnp.broadcast_to(fg[:, None, :], (Gb, Kn, 256)).reshape(R, 256)
    cat = jnp.concatenate([fgb, h2], axis=1)
    h3 = jnp.dot(cat, w3_ref[...], preferred_element_type=jnp.float32) + b3_ref[...]
    h3_ref[...] = h3.astype(jnp.bfloat16)
    ssum = jnp.sum(h3, axis=0, keepdims=True)
    ssq = jnp.sum(h3 * h3, axis=0, keepdims=True)
    acc = jnp.concatenate([ssum, ssq], axis=0)
    @pl.when(pl.program_id(0) == 0)
    def _():
        sums_ref[...] = acc
    @pl.when(pl.program_id(0) != 0)
    def _():
        sums_ref[...] += acc


def _encoder1(nbx, nby, nbz, cen, W1, s1, t1, W2, b2, W3, b3):
    BG = nbx.shape[0]
    M = BG * _K
    GB = 32
    n = BG // GB
    full = lambda shp: pl.BlockSpec(shp, lambda i: (0, 0))
    blk = lambda shp: pl.BlockSpec(shp, lambda i: (i, 0))
    return pl.pallas_call(
        _enc1_body,
        grid=(n,),
        in_specs=[
            blk((GB, _K)), blk((GB, _K)), blk((GB, _K)), blk((GB, 16)),
            full((3, 128)), full((1, 128)), full((1, 128)),
            full((128, 256)), full((1, 256)),
            full((512, 512)), full((1, 512)),
        ],
        out_specs=(pl.BlockSpec((GB * _K, 512), lambda i: (i, 0)),
                   pl.BlockSpec((2, 512), lambda i: (0, 0))),
        out_shape=(jax.ShapeDtypeStruct((M, 512), jnp.bfloat16),
                   jax.ShapeDtypeStruct((2, 512), jnp.float32)),
    )(nbx, nby, nbz, cen, W1, s1, t1, W2, b2, W3, b3)


def _enc2_body(h3_ref, sums_ref, g2_ref, be2_ref, w4_ref, b4_ref, feat_ref,
               *, M):
    h3 = h3_ref[...].astype(jnp.float32)
    R = h3.shape[0]
    Gb = R // _K
    mu = sums_ref[0:1, :] / M
    var = sums_ref[1:2, :] / M - mu * mu
    s2 = g2_ref[...] * lax.rsqrt(var + _EPS)
    t2 = be2_ref[...] - mu * s2
    a3 = jnp.maximum(h3 * s2 + t2, 0.0)
    h4 = jnp.dot(a3, w4_ref[...], preferred_element_type=jnp.float32) + b4_ref[...]
    feat_ref[...] = jnp.max(h4.reshape(Gb, _K, 256), axis=1)


def _encoder2(h3, sums, g2, be2, W4, b4):
    M = h3.shape[0]
    RB = 1024
    GB = RB // _K
    n = M // RB
    full = lambda shp: pl.BlockSpec(shp, lambda i: (0, 0))
    return pl.pallas_call(
        functools.partial(_enc2_body, M=float(M)),
        grid=(n,),
        in_specs=[
            pl.BlockSpec((RB, 512), lambda i: (i, 0)),
            full((2, 512)), full((1, 512)), full((1, 512)),
            full((512, 256)), full((1, 256)),
        ],
        out_specs=pl.BlockSpec((GB, 256), lambda i: (i, 0)),
        out_shape=jax.ShapeDtypeStruct((M // _K, 256), jnp.float32),
    )(h3, sums, g2, be2, W4, b4)


def _vq_body(feat_ref, cbt_ref, minv_ref, tok_ref):
    TB = cbt_ref.shape[1]
    cbt = cbt_ref[...]
    cn = jnp.sum(cbt * cbt, axis=0, keepdims=True)
    sc = jnp.dot(feat_ref[...], cbt,
                 preferred_element_type=jnp.float32)
    d2 = cn - 2.0 * sc
    m = jnp.min(d2, axis=1, keepdims=True)
    iota_t = lax.broadcasted_iota(jnp.int32, d2.shape, 1)
    a = jnp.min(jnp.where(d2 == m, iota_t, jnp.int32(2**30)), axis=1,
                keepdims=True) + pl.program_id(0) * TB
    @pl.when(pl.program_id(0) == 0)
    def _():
        minv_ref[...] = m
        tok_ref[...] = a
    @pl.when(pl.program_id(0) != 0)
    def _():
        better = m < minv_ref[...]
        minv_ref[...] = jnp.where(better, m, minv_ref[...])
        tok_ref[...] = jnp.where(better, a, tok_ref[...])


def _vq(feat, cbT):
    BG = feat.shape[0]
    T = cbT.shape[1]
    TB = 512
    return pl.pallas_call(
        _vq_body,
        grid=(T // TB,),
        in_specs=[
            pl.BlockSpec((BG, 256), lambda i: (0, 0)),
            pl.BlockSpec((256, TB), lambda i: (0, i)),
        ],
        out_specs=(pl.BlockSpec((BG, 1), lambda i: (0, 0)),
                   pl.BlockSpec((BG, 1), lambda i: (0, 0))),
        out_shape=(jax.ShapeDtypeStruct((BG, 1), jnp.float32),
                   jax.ShapeDtypeStruct((BG, 1), jnp.int32)),
    )(feat, cbT)


def _dec_body(q_ref, nbx_ref, nby_ref, nbz_ref, cen_ref, d1_ref, db1_ref,
              d2_ref, db2_ref, d3x_ref, d3y_ref, d3z_ref, db3x_ref,
              db3y_ref, db3z_ref, out_ref, *, denom):
    q = q_ref[...]
    h = jnp.maximum(jnp.dot(q, d1_ref[...], preferred_element_type=jnp.float32)
                    + db1_ref[...], 0.0)
    h = jnp.maximum(jnp.dot(h, d2_ref[...], preferred_element_type=jnp.float32)
                    + db2_ref[...], 0.0)
    rx = jnp.dot(h, d3x_ref[...], preferred_element_type=jnp.float32) + db3x_ref[...]
    ry = jnp.dot(h, d3y_ref[...], preferred_element_type=jnp.float32) + db3y_ref[...]
    rz = jnp.dot(h, d3z_ref[...], preferred_element_type=jnp.float32) + db3z_ref[...]
    gx = nbx_ref[...] - cen_ref[:, 0:1]
    gy = nby_ref[...] - cen_ref[:, 1:2]
    gz = nbz_ref[...] - cen_ref[:, 2:3]
    tx = rx[:, :, None] - gx[:, None, :]
    ty = ry[:, :, None] - gy[:, None, :]
    tz = rz[:, :, None] - gz[:, None, :]
    dd = tx * tx + ty * ty + tz * tz
    part = (jnp.sum(jnp.min(dd, axis=2)) + jnp.sum(jnp.min(dd, axis=1))) / denom
    part = jnp.reshape(part, (1, 1))
    @pl.when(pl.program_id(0) == 0)
    def _():
        out_ref[...] = part
    @pl.when(pl.program_id(0) != 0)
    def _():
        out_ref[...] += part


def _decode_chamfer(q, nbx, nby, nbz, cen, D1, db1, D2, db2,
                    D3x, D3y, D3z, db3x, db3y, db3z):
    BG = q.shape[0]
    GB = 256
    n = BG // GB
    denom = float(BG * _K)
    full = lambda shp: pl.BlockSpec(shp, lambda i: (0, 0))
    blk = lambda r: pl.BlockSpec(r, lambda i: (i, 0))
    return pl.pallas_call(
        functools.partial(_dec_body, denom=denom),
        grid=(n,),
        in_specs=[
            blk((GB, 256)), blk((GB, _K)), blk((GB, _K)), blk((GB, _K)),
            blk((GB, 16)),
            full((256, 512)), full((1, 512)),
            full((512, 256)), full((1, 256)),
            full((256, _K)), full((256, _K)), full((256, _K)),
            full((1, _K)), full((1, _K)), full((1, _K)),
        ],
        out_specs=pl.BlockSpec((1, 1), lambda i: (0, 0)),
        out_shape=jax.ShapeDtypeStruct((1, 1), jnp.float32),
    )(q, nbx, nby, nbz, cen, D1, db1, D2, db2,
      D3x, D3y, D3z, db3x, db3y, db3z)


def kernel(pts, W1, b1, g1, be1, W2, b2, W3, b3, g2, be2, W4, b4,
           codebook, D1, db1, D2, db2, D3, db3):
    B, N, _ = pts.shape
    C = W4.shape[1]
    BG = B * _G
    M = BG * _K

    px = pts[:, :, 0]
    py = pts[:, :, 1]
    pz = pts[:, :, 2]
    cen16, knn_kg = _fps_knn(px, py, pz)
    knn = knn_kg.transpose(0, 2, 1)

    nbx, nby, nbz = _sc_gather_nb(px, py, pz, knn.reshape(M))
    nbx = nbx.reshape(BG, _K)
    nby = nby.reshape(BG, _K)
    nbz = nbz.reshape(BG, _K)

    cen = cen16.reshape(BG, 16)
    r = lambda v: v.reshape(1, -1)
    s1, t1 = _bn1_stats(nbx, nby, nbz, cen, W1, r(b1), r(g1), r(be1))
    h3, sums = _encoder1(nbx, nby, nbz, cen, W1, s1, t1, W2, r(b2), W3, r(b3))
    feat = _encoder2(h3, sums, r(g2), r(be2), W4, r(b4))
    _, tok = _vq(feat, codebook.T)
    q = _sc_gather_rows(codebook, tok.reshape(BG), C)

    loss = _decode_chamfer(
        q, nbx, nby, nbz, cen, D1, r(db1), D2, r(db2),
        D3[:, 0::3], D3[:, 1::3], D3[:, 2::3],
        r(db3[0::3]), r(db3[1::3]), r(db3[2::3]))
    return loss[0, 0]
